# baseline (speedup 1.0000x reference)
import functools
import math

import jax
import jax.numpy as jnp
from jax import lax
from jax.experimental import pallas as pl
from jax.experimental.pallas import tpu as pltpu
from jax.experimental.pallas import tpu_sc as plsc

D_MODEL = 1024
L = 16


def _pe_table(T, d_model):
    pos = jnp.arange(T, dtype=jnp.float32)[:, None]
    div_term = jnp.exp(
        jnp.arange(0, d_model, 2, dtype=jnp.float32) * (-math.log(10000.0) / d_model)
    )
    pe = jnp.zeros((T, d_model), dtype=jnp.float32)
    pe = pe.at[:, 0::2].set(jnp.sin(pos * div_term))
    pe = pe.at[:, 1::2].set(jnp.cos(pos * div_term))
    return pe


@functools.partial(jax.jit, static_argnames=("B", "T", "D"))
def _sc_embed_add(x_flat3, emb_table, pe, *, B, T, D):
    N = B * T
    info = plsc.get_sparse_core_info()
    NC, NS = info.num_cores, info.num_subcores
    NW = NC * NS
    rows_per_w = N // NW
    CHUNK = 16
    n_chunks = rows_per_w // CHUNK

    mesh = plsc.VectorSubcoreMesh(core_axis_name="c", subcore_axis_name="s")

    @functools.partial(
        pl.kernel,
        mesh=mesh,
        out_type=jax.ShapeDtypeStruct((N, D), jnp.float32),
        scratch_types=[
            pltpu.VMEM((n_chunks, CHUNK), jnp.int32),
            pltpu.VMEM((CHUNK, D), jnp.float32),
            pltpu.VMEM((CHUNK, D), jnp.float32),
            pltpu.SemaphoreType.DMA,
            pltpu.SemaphoreType.DMA,
        ],
    )
    def k(idx_hbm, table_hbm, pe_hbm, out_hbm, idx_v, rows_v, pe_v, gsem, psem):
        wid = lax.axis_index("s") * NC + lax.axis_index("c")
        base = wid * rows_per_w
        t_base = base % T
        pltpu.sync_copy(idx_hbm.at[wid], idx_v)

        def chunk_body(c, _):
            g = pltpu.async_copy(table_hbm.at[idx_v.at[c]], rows_v, gsem)
            p = pltpu.async_copy(pe_hbm.at[pl.ds(t_base + c * CHUNK, CHUNK)], pe_v, psem)
            g.wait()
            p.wait()

            def col_body(j, _):
                for r in range(CHUNK):
                    rows_v[r, pl.ds(j * L, L)] = (
                        rows_v[r, pl.ds(j * L, L)] + pe_v[r, pl.ds(j * L, L)]
                    )
                return 0

            lax.fori_loop(0, D // L, col_body, 0, unroll=2)
            pltpu.sync_copy(rows_v, out_hbm.at[pl.ds(base + c * CHUNK, CHUNK)])
            return 0

        lax.fori_loop(0, n_chunks, chunk_body, 0)

    out = k(x_flat3, emb_table, pe)
    return out


def kernel(x, emb_table):
    B, T = x.shape
    V, D = emb_table.shape
    N = B * T
    NW = 32
    rows_per_w = N // NW
    CHUNK = 16
    pe = _pe_table(T, D)
    x_flat3 = x.astype(jnp.int32).reshape(NW, rows_per_w // CHUNK, CHUNK)
    out = _sc_embed_add(x_flat3, emb_table, pe, B=B, T=T, D=D)
    return out.reshape(B, T, D)
